# baseline (speedup 1.0000x reference)
import jax
import jax.numpy as jnp
from jax import lax
from jax.experimental import pallas as pl
from jax.experimental.pallas import tpu as pltpu
from jax.experimental.pallas import tpu_sc as plsc

N_ENT = 100000
RANK = 32
BATCH = 64
CHUNK = 16
BLK = 1024
TRB = 256
N_TRB = (N_ENT + TRB - 1) // TRB
N_WORKERS = 32


def _sc_prep_body(lhs_w, rel_w, ix_l, ix_r, ix_o,
                  lhs_o, rel_o, rhs_o, tab_t_o,
                  idx_v, grows_v, rows_v, colblk_v, sem):
    nc = plsc.get_sparse_core_info().num_cores
    wid = lax.axis_index("s") * nc + lax.axis_index("c")
    table = wid // 4
    base = (wid % 4) * CHUNK

    @pl.when(table == 0)
    def _():
        pltpu.sync_copy(ix_l.at[pl.ds(base, CHUNK)], idx_v)
        pltpu.async_copy(lhs_w.at[idx_v], grows_v, sem).wait()
        pltpu.sync_copy(grows_v, lhs_o.at[pl.ds(base, CHUNK)])

    @pl.when(table == 1)
    def _():
        pltpu.sync_copy(ix_r.at[pl.ds(base, CHUNK)], idx_v)
        pltpu.async_copy(rel_w.at[idx_v], grows_v, sem).wait()
        pltpu.sync_copy(grows_v, rel_o.at[pl.ds(base, CHUNK)])

    @pl.when(table == 2)
    def _():
        pltpu.sync_copy(ix_o.at[pl.ds(base, CHUNK)], idx_v)
        pltpu.async_copy(lhs_w.at[idx_v], grows_v, sem).wait()
        pltpu.sync_copy(grows_v, rhs_o.at[pl.ds(base, CHUNK)])

    n_k = (N_TRB + N_WORKERS - 1) // N_WORKERS

    def blk_body(k, carry):
        blk = wid + N_WORKERS * k

        @pl.when(blk < N_TRB)
        def _():
            row0 = jnp.minimum(blk * TRB, N_ENT - TRB)
            pltpu.sync_copy(lhs_w.at[pl.ds(row0, TRB)], rows_v)

            def col_body(c, inner):
                for g in range(TRB // 16):
                    ridx = g * 16 + lax.iota(jnp.int32, 16)
                    cidx = jnp.full((16,), c, jnp.int32)
                    vals = plsc.load_gather(rows_v, [ridx, cidx])
                    colblk_v[c, pl.ds(g * 16, 16)] = vals
                return inner

            lax.fori_loop(0, RANK, col_body, 0)
            pltpu.sync_copy(colblk_v, tab_t_o.at[:, pl.ds(row0, TRB)])

        return carry

    lax.fori_loop(0, n_k, blk_body, 0)


def _sc_prep(lhs_weight, rel_weight, ix_l, ix_r, ix_o):
    emb = jax.ShapeDtypeStruct((BATCH, RANK), jnp.float32)
    tab_t = jax.ShapeDtypeStruct((RANK, N_ENT), jnp.float32)
    run = pl.kernel(
        _sc_prep_body,
        out_type=(emb, emb, emb, tab_t),
        mesh=plsc.VectorSubcoreMesh(core_axis_name="c", subcore_axis_name="s"),
        scratch_types=[
            pltpu.VMEM((CHUNK,), jnp.int32),
            pltpu.VMEM((CHUNK, RANK), jnp.float32),
            pltpu.VMEM((TRB, RANK), jnp.float32),
            pltpu.VMEM((RANK, TRB), jnp.float32),
            pltpu.SemaphoreType.DMA,
        ],
        compiler_params=pltpu.CompilerParams(
            use_tc_tiling_on_sc=False, needs_layout_passes=False),
    )
    return run(lhs_weight, rel_weight, ix_l, ix_r, ix_o)


def _score_body(tab_t_ref, lhs_ref, rel_ref, rhs_ref, sp_ref, po_ref):
    q1 = (lhs_ref[...] + rel_ref[...]).astype(jnp.bfloat16)
    q2 = (rhs_ref[...] - rel_ref[...]).astype(jnp.bfloat16)
    t = tab_t_ref[...].astype(jnp.bfloat16)
    acc1 = jnp.zeros(sp_ref.shape, jnp.float32)
    acc2 = jnp.zeros(po_ref.shape, jnp.float32)
    for g in range(RANK // 8):
        p1 = jnp.zeros(sp_ref.shape, jnp.bfloat16)
        p2 = jnp.zeros(po_ref.shape, jnp.bfloat16)
        for r in range(g * 8, g * 8 + 8):
            tr = t[r:r + 1, :]
            p1 = p1 - jnp.abs(q1[:, r:r + 1] - tr)
            p2 = p2 - jnp.abs(q2[:, r:r + 1] - tr)
        acc1 = acc1 + p1.astype(jnp.float32)
        acc2 = acc2 + p2.astype(jnp.float32)
    sp_ref[...] = acc1
    po_ref[...] = acc2


def _tc_score(tab_t, lhs, rel, rhs):
    n_ent = tab_t.shape[1]
    grid = (pl.cdiv(n_ent, BLK),)
    out = jax.ShapeDtypeStruct((BATCH, n_ent), jnp.float32)
    scores = pl.pallas_call(
        _score_body,
        grid=grid,
        in_specs=[
            pl.BlockSpec((RANK, BLK), lambda i: (0, i)),
            pl.BlockSpec((BATCH, RANK), lambda i: (0, 0)),
            pl.BlockSpec((BATCH, RANK), lambda i: (0, 0)),
            pl.BlockSpec((BATCH, RANK), lambda i: (0, 0)),
        ],
        out_specs=[
            pl.BlockSpec((BATCH, BLK), lambda i: (0, i)),
            pl.BlockSpec((BATCH, BLK), lambda i: (0, i)),
        ],
        out_shape=[out, out],
    )(tab_t, lhs, rel, rhs)
    return scores


@jax.jit
def kernel(x, lhs_weight, rel_weight):
    ix_l = x[:, 0]
    ix_r = x[:, 1]
    ix_o = x[:, 2]
    lhs, rel, rhs, tab_t = _sc_prep(lhs_weight, rel_weight, ix_l, ix_r, ix_o)
    scores_sp, scores_po = _tc_score(tab_t, lhs, rel, rhs)
    return (scores_sp, scores_po, (lhs, rel, rhs))


# speedup vs baseline: 1.5640x; 1.5640x over previous
import jax
import jax.numpy as jnp
from jax import lax
from jax.experimental import pallas as pl
from jax.experimental.pallas import tpu as pltpu
from jax.experimental.pallas import tpu_sc as plsc

RANK = 32
BATCH = 64
CHUNK = 16
BLK = 1024


def _sc_gather_body(lhs_w, rel_w, ix_l, ix_r, ix_o,
                    lhs_o, rel_o, rhs_o, idx_v, rows_v, sem):
    nc = plsc.get_sparse_core_info().num_cores
    wid = lax.axis_index("s") * nc + lax.axis_index("c")
    table = wid // 4
    base = (wid % 4) * CHUNK

    @pl.when(table == 0)
    def _():
        pltpu.sync_copy(ix_l.at[pl.ds(base, CHUNK)], idx_v)
        pltpu.async_copy(lhs_w.at[idx_v], rows_v, sem).wait()
        pltpu.sync_copy(rows_v, lhs_o.at[pl.ds(base, CHUNK)])

    @pl.when(table == 1)
    def _():
        pltpu.sync_copy(ix_r.at[pl.ds(base, CHUNK)], idx_v)
        pltpu.async_copy(rel_w.at[idx_v], rows_v, sem).wait()
        pltpu.sync_copy(rows_v, rel_o.at[pl.ds(base, CHUNK)])

    @pl.when(table == 2)
    def _():
        pltpu.sync_copy(ix_o.at[pl.ds(base, CHUNK)], idx_v)
        pltpu.async_copy(lhs_w.at[idx_v], rows_v, sem).wait()
        pltpu.sync_copy(rows_v, rhs_o.at[pl.ds(base, CHUNK)])


def _sc_gather(lhs_weight, rel_weight, ix_l, ix_r, ix_o):
    emb = jax.ShapeDtypeStruct((BATCH, RANK), jnp.float32)
    run = pl.kernel(
        _sc_gather_body,
        out_type=(emb, emb, emb),
        mesh=plsc.VectorSubcoreMesh(core_axis_name="c", subcore_axis_name="s"),
        scratch_types=[
            pltpu.VMEM((CHUNK,), jnp.int32),
            pltpu.VMEM((CHUNK, RANK), jnp.float32),
            pltpu.SemaphoreType.DMA,
        ],
        compiler_params=pltpu.CompilerParams(use_tc_tiling_on_sc=False),
    )
    return run(lhs_weight, rel_weight, ix_l, ix_r, ix_o)


def _score_body(tab_t_ref, lhs_ref, rel_ref, rhs_ref, sp_ref, po_ref):
    q1 = (lhs_ref[...] + rel_ref[...]).astype(jnp.bfloat16)
    q2 = (rhs_ref[...] - rel_ref[...]).astype(jnp.bfloat16)
    t = tab_t_ref[...]
    acc1 = jnp.zeros(sp_ref.shape, jnp.float32)
    acc2 = jnp.zeros(po_ref.shape, jnp.float32)
    for g in range(RANK // 8):
        p1 = jnp.zeros(sp_ref.shape, jnp.bfloat16)
        p2 = jnp.zeros(po_ref.shape, jnp.bfloat16)
        for r in range(g * 8, g * 8 + 8):
            tr = t[r:r + 1, :]
            p1 = p1 - jnp.abs(q1[:, r:r + 1] - tr)
            p2 = p2 - jnp.abs(q2[:, r:r + 1] - tr)
        acc1 = acc1 + p1.astype(jnp.float32)
        acc2 = acc2 + p2.astype(jnp.float32)
    sp_ref[...] = acc1
    po_ref[...] = acc2


def _tc_score(tab_t, lhs, rel, rhs):
    n_ent = tab_t.shape[1]
    grid = (pl.cdiv(n_ent, BLK),)
    out = jax.ShapeDtypeStruct((BATCH, n_ent), jnp.float32)
    scores = pl.pallas_call(
        _score_body,
        grid=grid,
        in_specs=[
            pl.BlockSpec((RANK, BLK), lambda i: (0, i)),
            pl.BlockSpec((BATCH, RANK), lambda i: (0, 0)),
            pl.BlockSpec((BATCH, RANK), lambda i: (0, 0)),
            pl.BlockSpec((BATCH, RANK), lambda i: (0, 0)),
        ],
        out_specs=[
            pl.BlockSpec((BATCH, BLK), lambda i: (0, i)),
            pl.BlockSpec((BATCH, BLK), lambda i: (0, i)),
        ],
        out_shape=[out, out],
    )(tab_t, lhs, rel, rhs)
    return scores


@jax.jit
def kernel(x, lhs_weight, rel_weight):
    ix_l = x[:, 0]
    ix_r = x[:, 1]
    ix_o = x[:, 2]
    lhs, rel, rhs = _sc_gather(lhs_weight, rel_weight, ix_l, ix_r, ix_o)
    tab_t = lhs_weight.astype(jnp.bfloat16).T
    scores_sp, scores_po = _tc_score(tab_t, lhs, rel, rhs)
    return (scores_sp, scores_po, (lhs, rel, rhs))
